# baseline (speedup 1.0000x reference)
import functools

import jax
import jax.numpy as jnp
from jax.experimental import pallas as pl
from jax.experimental.pallas import tpu as pltpu

INPUT_DIM = 512
NUM_EMB = 8192
EMB_DIM = 64
NUM_SPLITS = 8
BATCH = 4096

BT = 128


def _vq_kernel(x_ref, We_ref, be_ref, cb_ref, Wd_ref, bd_ref,
               xr_ref, q_ref, codes_ref, perp_ref, counts_ref):
    b = pl.program_id(0)
    nb = pl.num_programs(0)

    @pl.when(b == 0)
    def _init():
        counts_ref[...] = jnp.zeros_like(counts_ref)

    x = x_ref[...]
    z = jnp.dot(x, We_ref[...]) + be_ref[...]

    quant_cols = []
    idx_cols = []
    for s in range(NUM_SPLITS):
        flat = z[:, s * EMB_DIM:(s + 1) * EMB_DIM]
        Es = cb_ref[s]
        m = jax.lax.dot_general(flat, Es,
                                (((1,), (1,)), ((), ())))
        s_flat = jnp.sum(flat * flat, axis=1, keepdims=True)
        s_E = jnp.sum(Es * Es, axis=1)
        d = (s_flat + s_E) - 2.0 * m
        dmin = jnp.min(d, axis=1, keepdims=True)
        iota = jax.lax.broadcasted_iota(jnp.int32, d.shape, 1)
        idx = jnp.min(jnp.where(d == dmin, iota, NUM_EMB),
                      axis=1, keepdims=True)
        onehot = (iota == idx).astype(jnp.float32)
        qs = jax.lax.dot(onehot, Es,
                         precision=jax.lax.Precision.HIGHEST)
        quant_cols.append(qs)
        idx_cols.append(idx)
        counts_ref[s, :] = counts_ref[s, :] + jnp.sum(onehot, axis=0)

    q = jnp.concatenate(quant_cols, axis=1)
    q_ref[...] = q
    codes_ref[...] = jnp.concatenate(idx_cols, axis=1)
    xr_ref[...] = jnp.dot(q, Wd_ref[...]) + bd_ref[...]

    @pl.when(b == nb - 1)
    def _finish():
        avg = counts_ref[...] * (1.0 / BATCH)
        plogp = avg * jnp.log(avg + 1e-10)
        ent = jnp.sum(plogp, axis=1, keepdims=True)
        perps = jnp.exp(-ent)
        val = jnp.sum(perps) * (1.0 / NUM_SPLITS)
        perp_ref[...] = jnp.full((1, 128), val, dtype=jnp.float32)


@functools.partial(jax.jit, static_argnames=())
def kernel(x, W_enc, b_enc, codebooks, W_dec, b_dec):
    lat = EMB_DIM * NUM_SPLITS
    nb = BATCH // BT
    be2 = b_enc.reshape(1, lat)
    bd2 = b_dec.reshape(1, INPUT_DIM)
    out_shapes = (
        jax.ShapeDtypeStruct((BATCH, INPUT_DIM), jnp.float32),
        jax.ShapeDtypeStruct((BATCH, lat), jnp.float32),
        jax.ShapeDtypeStruct((BATCH, NUM_SPLITS), jnp.int32),
        jax.ShapeDtypeStruct((1, 128), jnp.float32),
        jax.ShapeDtypeStruct((NUM_SPLITS, NUM_EMB), jnp.float32),
    )
    grid_spec = pl.GridSpec(
        grid=(nb,),
        in_specs=[
            pl.BlockSpec((BT, INPUT_DIM), lambda b: (b, 0)),
            pl.BlockSpec((INPUT_DIM, lat), lambda b: (0, 0)),
            pl.BlockSpec((1, lat), lambda b: (0, 0)),
            pl.BlockSpec((NUM_SPLITS, NUM_EMB, EMB_DIM), lambda b: (0, 0, 0)),
            pl.BlockSpec((lat, INPUT_DIM), lambda b: (0, 0)),
            pl.BlockSpec((1, INPUT_DIM), lambda b: (0, 0)),
        ],
        out_specs=(
            pl.BlockSpec((BT, INPUT_DIM), lambda b: (b, 0)),
            pl.BlockSpec((BT, lat), lambda b: (b, 0)),
            pl.BlockSpec((BT, NUM_SPLITS), lambda b: (b, 0)),
            pl.BlockSpec((1, 128), lambda b: (0, 0)),
            pl.BlockSpec((NUM_SPLITS, NUM_EMB), lambda b: (0, 0)),
        ),
    )
    x_recon, quantized, codes, perp, _counts = pl.pallas_call(
        _vq_kernel,
        grid_spec=grid_spec,
        out_shape=out_shapes,
    )(x, W_enc, be2, codebooks, W_dec, bd2)
    return x_recon, quantized, codes, perp[0, 0]


# speedup vs baseline: 2.1487x; 2.1487x over previous
import functools

import jax
import jax.numpy as jnp
from jax.experimental import pallas as pl
from jax.experimental.pallas import tpu as pltpu

INPUT_DIM = 512
NUM_EMB = 8192
EMB_DIM = 64
NUM_SPLITS = 8
BATCH = 4096
LAT = EMB_DIM * NUM_SPLITS

BT = 128


def _vq_kernel(x_ref, We_ref, be_ref, cbs_ref, Wd_ref, bd_ref,
               xr_ref, q_ref, codes_ref, perp_ref, counts_ref,
               z_scr, xr_scr, q_scr, codes_scr, iota_scr):
    s = pl.program_id(0)
    b = pl.program_id(1)
    nb = pl.num_programs(1)
    row0 = b * BT

    @pl.when((s == 0) & (b == 0))
    def _init():
        counts_ref[...] = jnp.zeros_like(counts_ref)
        iota_scr[...] = jax.lax.broadcasted_iota(jnp.int32, (BT, NUM_EMB), 1)

    @pl.when(s == 0)
    def _encode():
        z = jnp.dot(x_ref[...], We_ref[...]) + be_ref[...]
        for ss in range(NUM_SPLITS):
            z_scr[ss, pl.ds(row0, BT), :] = z[:, ss * EMB_DIM:(ss + 1) * EMB_DIM]

    cs = cbs_ref[0]
    sE = (jnp.sum(cs * cs, axis=1) * 0.25).reshape(1, NUM_EMB)

    flat = z_scr[s, pl.ds(row0, BT), :]
    m2 = jax.lax.dot_general(flat, cs,
                             (((1,), (1,)), ((), ())))
    s_flat = jnp.sum(flat * flat, axis=1, keepdims=True)
    d = (s_flat + sE) + m2
    dmin = jnp.min(d, axis=1, keepdims=True)
    iota = iota_scr[...]
    key = jnp.where(d == dmin, iota, NUM_EMB)
    idx = jnp.min(key, axis=1, keepdims=True)
    onehot = jnp.where(key == idx, 1.0, 0.0)
    qs = jnp.dot(onehot, cs) * -0.5

    lane_grp = jax.lax.broadcasted_iota(jnp.int32, (BT, LAT), 1) // EMB_DIM
    q_band = jnp.concatenate([qs] * NUM_SPLITS, axis=1)
    old_q = q_scr[pl.ds(row0, BT), :]
    q_scr[pl.ds(row0, BT), :] = jnp.where(lane_grp == s, q_band, old_q)

    cgrp = jax.lax.broadcasted_iota(jnp.int32, (BT, NUM_SPLITS), 1)
    old_c = codes_scr[pl.ds(row0, BT), :]
    codes_scr[pl.ds(row0, BT), :] = jnp.where(cgrp == s, idx, old_c)

    colsum = jnp.sum(onehot, axis=0).reshape(1, NUM_EMB)
    rgrp = jax.lax.broadcasted_iota(jnp.int32, (NUM_SPLITS, NUM_EMB), 0)
    counts_ref[...] = counts_ref[...] + jnp.where(rgrp == s, colsum, 0.0)

    part = jnp.dot(qs, Wd_ref[0])

    @pl.when(s == 0)
    def _dec0():
        xr_scr[pl.ds(row0, BT), :] = part

    @pl.when(s > 0)
    def _dec():
        xr_scr[pl.ds(row0, BT), :] = xr_scr[pl.ds(row0, BT), :] + part

    @pl.when(s == NUM_SPLITS - 1)
    def _emit():
        xr_ref[...] = xr_scr[pl.ds(row0, BT), :] + bd_ref[...]
        q_ref[...] = q_scr[pl.ds(row0, BT), :]
        codes_ref[...] = codes_scr[pl.ds(row0, BT), :]

    @pl.when((s == NUM_SPLITS - 1) & (b == nb - 1))
    def _finish():
        avg = counts_ref[...] * (1.0 / BATCH)
        plogp = avg * jnp.log(avg + 1e-10)
        ent = jnp.sum(plogp, axis=1, keepdims=True)
        perps = jnp.exp(-ent)
        val = jnp.sum(perps) * (1.0 / NUM_SPLITS)
        perp_ref[...] = jnp.full((1, 128), val, dtype=jnp.float32)


@functools.partial(jax.jit, static_argnames=())
def kernel(x, W_enc, b_enc, codebooks, W_dec, b_dec):
    nb = BATCH // BT
    be2 = b_enc.reshape(1, LAT)
    bd2 = b_dec.reshape(1, INPUT_DIM)
    cbs = codebooks * (-2.0)
    Wd3 = W_dec.reshape(NUM_SPLITS, EMB_DIM, INPUT_DIM)
    out_shapes = (
        jax.ShapeDtypeStruct((BATCH, INPUT_DIM), jnp.float32),
        jax.ShapeDtypeStruct((BATCH, LAT), jnp.float32),
        jax.ShapeDtypeStruct((BATCH, NUM_SPLITS), jnp.int32),
        jax.ShapeDtypeStruct((1, 128), jnp.float32),
        jax.ShapeDtypeStruct((NUM_SPLITS, NUM_EMB), jnp.float32),
    )
    grid_spec = pltpu.PrefetchScalarGridSpec(
        num_scalar_prefetch=0,
        grid=(NUM_SPLITS, nb),
        scratch_shapes=[
            pltpu.VMEM((NUM_SPLITS, BATCH, EMB_DIM), jnp.float32),
            pltpu.VMEM((BATCH, INPUT_DIM), jnp.float32),
            pltpu.VMEM((BATCH, LAT), jnp.float32),
            pltpu.VMEM((BATCH, NUM_SPLITS), jnp.int32),
            pltpu.VMEM((BT, NUM_EMB), jnp.int32),
        ],
        in_specs=[
            pl.BlockSpec((BT, INPUT_DIM), lambda s, b: (b, 0)),
            pl.BlockSpec((INPUT_DIM, LAT), lambda s, b: (0, 0)),
            pl.BlockSpec((1, LAT), lambda s, b: (0, 0)),
            pl.BlockSpec((1, NUM_EMB, EMB_DIM), lambda s, b: (s, 0, 0)),
            pl.BlockSpec((1, EMB_DIM, INPUT_DIM), lambda s, b: (s, 0, 0)),
            pl.BlockSpec((1, INPUT_DIM), lambda s, b: (0, 0)),
        ],
        out_specs=(
            pl.BlockSpec((BT, INPUT_DIM), lambda s, b: (b, 0)),
            pl.BlockSpec((BT, LAT), lambda s, b: (b, 0)),
            pl.BlockSpec((BT, NUM_SPLITS), lambda s, b: (b, 0)),
            pl.BlockSpec((1, 128), lambda s, b: (0, 0)),
            pl.BlockSpec((NUM_SPLITS, NUM_EMB), lambda s, b: (0, 0)),
        ),
    )
    x_recon, quantized, codes, perp, _counts = pl.pallas_call(
        _vq_kernel,
        grid_spec=grid_spec,
        out_shape=out_shapes,
    )(x, W_enc, be2, cbs, Wd3, bd2)
    return x_recon, quantized, codes, perp[0, 0]


# speedup vs baseline: 2.2230x; 1.0346x over previous
import functools

import jax
import jax.numpy as jnp
from jax.experimental import pallas as pl
from jax.experimental.pallas import tpu as pltpu

INPUT_DIM = 512
NUM_EMB = 8192
EMB_DIM = 64
NUM_SPLITS = 8
BATCH = 4096
LAT = EMB_DIM * NUM_SPLITS

BT = 128


def _vq_kernel(x_ref, We_ref, be_ref, cbs_ref, Wd_ref, bd_ref,
               xr_ref, q_ref, codes_ref, perp_ref, counts_ref,
               z_scr, xr_scr, codes_scr, sE_scr):
    s = pl.program_id(0)
    b = pl.program_id(1)
    nb = pl.num_programs(1)
    row0 = b * BT

    @pl.when((s == 0) & (b == 0))
    def _init():
        counts_ref[...] = jnp.zeros_like(counts_ref)

    @pl.when(s == 0)
    def _encode():
        z = jnp.dot(x_ref[...], We_ref[...]) + be_ref[...]
        for ss in range(NUM_SPLITS):
            z_scr[ss, pl.ds(row0, BT), :] = z[:, ss * EMB_DIM:(ss + 1) * EMB_DIM]

    cs = cbs_ref[0]

    @pl.when(b == 0)
    def _se():
        ones = jnp.ones((1, EMB_DIM), jnp.float32)
        sE_scr[...] = jax.lax.dot_general(
            ones, cs * cs, (((1,), (1,)), ((), ())),
            precision=jax.lax.Precision.HIGHEST) * 0.25

    flat = z_scr[s, pl.ds(row0, BT), :]
    m2 = jax.lax.dot_general(flat, cs,
                             (((1,), (1,)), ((), ())))
    s_flat = jnp.sum(flat * flat, axis=1, keepdims=True)
    d = (s_flat + sE_scr[...]) + m2
    dmin = jnp.min(d, axis=1, keepdims=True)
    iota = jax.lax.broadcasted_iota(
        jnp.int32, (BT, NUM_EMB), 1).astype(jnp.float32)
    key = jnp.where(d == dmin, iota, float(NUM_EMB))
    idxf = jnp.min(key, axis=1, keepdims=True)
    idx = idxf.astype(jnp.int32)
    onehot = jnp.where(key == idxf, 1.0, 0.0)
    qs = jnp.dot(onehot, cs) * -0.5
    q_ref[...] = qs.reshape(1, BT, EMB_DIM)

    cgrp = jax.lax.broadcasted_iota(jnp.int32, (BT, NUM_SPLITS), 1)
    old_c = codes_scr[pl.ds(row0, BT), :]
    codes_scr[pl.ds(row0, BT), :] = jnp.where(cgrp == s, idx, old_c)

    colsum = jnp.sum(onehot, axis=0).reshape(1, NUM_EMB)
    rgrp = jax.lax.broadcasted_iota(jnp.int32, (NUM_SPLITS, NUM_EMB), 0)
    counts_ref[...] = counts_ref[...] + jnp.where(rgrp == s, colsum, 0.0)

    part = jnp.dot(qs, Wd_ref[0])

    @pl.when(s == 0)
    def _dec0():
        xr_scr[pl.ds(row0, BT), :] = part

    @pl.when(s > 0)
    def _dec():
        xr_scr[pl.ds(row0, BT), :] = xr_scr[pl.ds(row0, BT), :] + part

    @pl.when(s == NUM_SPLITS - 1)
    def _emit():
        xr_ref[...] = xr_scr[pl.ds(row0, BT), :] + bd_ref[...]
        codes_ref[...] = codes_scr[pl.ds(row0, BT), :]

    @pl.when((s == NUM_SPLITS - 1) & (b == nb - 1))
    def _finish():
        avg = counts_ref[...] * (1.0 / BATCH)
        plogp = avg * jnp.log(avg + 1e-10)
        ent = jnp.sum(plogp, axis=1, keepdims=True)
        perps = jnp.exp(-ent)
        val = jnp.sum(perps) * (1.0 / NUM_SPLITS)
        perp_ref[...] = jnp.full((1, 128), val, dtype=jnp.float32)


@functools.partial(jax.jit, static_argnames=())
def kernel(x, W_enc, b_enc, codebooks, W_dec, b_dec):
    nb = BATCH // BT
    be2 = b_enc.reshape(1, LAT)
    bd2 = b_dec.reshape(1, INPUT_DIM)
    cbs = codebooks * (-2.0)
    Wd3 = W_dec.reshape(NUM_SPLITS, EMB_DIM, INPUT_DIM)
    out_shapes = (
        jax.ShapeDtypeStruct((BATCH, INPUT_DIM), jnp.float32),
        jax.ShapeDtypeStruct((NUM_SPLITS, BATCH, EMB_DIM), jnp.float32),
        jax.ShapeDtypeStruct((BATCH, NUM_SPLITS), jnp.int32),
        jax.ShapeDtypeStruct((1, 128), jnp.float32),
        jax.ShapeDtypeStruct((NUM_SPLITS, NUM_EMB), jnp.float32),
    )
    grid_spec = pltpu.PrefetchScalarGridSpec(
        num_scalar_prefetch=0,
        grid=(NUM_SPLITS, nb),
        scratch_shapes=[
            pltpu.VMEM((NUM_SPLITS, BATCH, EMB_DIM), jnp.float32),
            pltpu.VMEM((BATCH, INPUT_DIM), jnp.float32),
            pltpu.VMEM((BATCH, NUM_SPLITS), jnp.int32),
            pltpu.VMEM((1, NUM_EMB), jnp.float32),
        ],
        in_specs=[
            pl.BlockSpec((BT, INPUT_DIM), lambda s, b: (b, 0)),
            pl.BlockSpec((INPUT_DIM, LAT), lambda s, b: (0, 0)),
            pl.BlockSpec((1, LAT), lambda s, b: (0, 0)),
            pl.BlockSpec((1, NUM_EMB, EMB_DIM), lambda s, b: (s, 0, 0)),
            pl.BlockSpec((1, EMB_DIM, INPUT_DIM), lambda s, b: (s, 0, 0)),
            pl.BlockSpec((1, INPUT_DIM), lambda s, b: (0, 0)),
        ],
        out_specs=(
            pl.BlockSpec((BT, INPUT_DIM), lambda s, b: (b, 0)),
            pl.BlockSpec((1, BT, EMB_DIM), lambda s, b: (s, b, 0)),
            pl.BlockSpec((BT, NUM_SPLITS), lambda s, b: (b, 0)),
            pl.BlockSpec((1, 128), lambda s, b: (0, 0)),
            pl.BlockSpec((NUM_SPLITS, NUM_EMB), lambda s, b: (0, 0)),
        ),
    )
    x_recon, q_sm, codes, perp, _counts = pl.pallas_call(
        _vq_kernel,
        grid_spec=grid_spec,
        out_shape=out_shapes,
    )(x, W_enc, be2, cbs, Wd3, bd2)
    quantized = q_sm.transpose(1, 0, 2).reshape(BATCH, LAT)
    return x_recon, quantized, codes, perp[0, 0]
